# baseline (speedup 1.0000x reference)
import functools

import jax
import jax.numpy as jnp
from jax import lax
from jax.experimental import pallas as pl
from jax.experimental.pallas import tpu as pltpu
from jax.experimental.pallas import tpu_sc as plsc

F = 26
V = 100000
D = 32
B = 16384
NC, NS = 2, 16
NW = NC * NS
NR = F * D
RPW = NR // NW
ICH = 4096
NCH = B // ICH

_mesh = plsc.VectorSubcoreMesh(
    core_axis_name="c", subcore_axis_name="s", num_cores=NC, num_subcores=NS
)


@functools.partial(
    pl.kernel,
    out_type=jax.ShapeDtypeStruct((NR, B), jnp.float32),
    mesh=_mesh,
    scratch_types=[
        pltpu.VMEM((V,), jnp.float32),
        pltpu.VMEM((2, ICH), jnp.int32),
        pltpu.VMEM((B,), jnp.float32),
        pltpu.SemaphoreType.DMA,
        pltpu.SemaphoreType.DMA,
        pltpu.SemaphoreType.DMA,
    ],
    compiler_params=pltpu.CompilerParams(needs_layout_passes=False),
)
def _emb(tab, idx, out, row_v, idx_v, dst_v, sem_r, sem_i, sem_o):
    wid = lax.axis_index("s") * NC + lax.axis_index("c")
    j0 = wid * RPW

    @pl.loop(0, RPW)
    def _row(r):
        j = j0 + r
        f = j // D
        d = lax.rem(j, D)
        cp_r = pltpu.async_copy(tab.at[f, d, pl.ds(0, V)], row_v, sem_r)
        cp_i = [None, None]
        cp_i[0] = pltpu.async_copy(idx.at[f, pl.ds(0, ICH)], idx_v.at[0], sem_i)

        @pl.when(r > 0)
        def _():
            pltpu.make_async_copy(dst_v, out.at[j, pl.ds(0, B)], sem_o).wait()

        cp_r.wait()
        for c in range(NCH):
            buf = c % 2
            cp_i[buf].wait()
            if c + 1 < NCH:
                cp_i[1 - buf] = pltpu.async_copy(
                    idx.at[f, pl.ds((c + 1) * ICH, ICH)], idx_v.at[1 - buf], sem_i
                )

            @plsc.parallel_loop(0, ICH // 16, unroll=16)
            def _g(t):
                iv = idx_v[buf, pl.ds(t * 16, 16)]
                dst_v[pl.ds(c * ICH + t * 16, 16)] = plsc.load_gather(row_v, [iv])

        pltpu.async_copy(dst_v, out.at[j, pl.ds(0, B)], sem_o)

    pltpu.make_async_copy(
        dst_v, out.at[NR - 1, pl.ds(0, B)], sem_o
    ).wait()


def kernel(index_list, tables):
    tab_t = tables.transpose(0, 2, 1)
    idx_t = index_list.astype(jnp.int32).T
    out_t = _emb(tab_t, idx_t)
    return out_t.T


# speedup vs baseline: 1.0031x; 1.0031x over previous
import functools

import jax
import jax.numpy as jnp
from jax import lax
from jax.experimental import pallas as pl
from jax.experimental.pallas import tpu as pltpu
from jax.experimental.pallas import tpu_sc as plsc

F = 26
V = 100000
D = 32
B = 16384
NC, NS = 2, 16
NW = NC * NS
NR = F * D
RPW = NR // NW
ICH = 4096
NCH = B // ICH

_mesh = plsc.VectorSubcoreMesh(
    core_axis_name="c", subcore_axis_name="s", num_cores=NC, num_subcores=NS
)


@functools.partial(
    pl.kernel,
    out_type=jax.ShapeDtypeStruct((NR, B), jnp.float32),
    mesh=_mesh,
    scratch_types=[
        pltpu.VMEM((V,), jnp.float32),
        pltpu.VMEM((2, ICH), jnp.int32),
        pltpu.VMEM((B,), jnp.float32),
        pltpu.SemaphoreType.DMA,
        pltpu.SemaphoreType.DMA,
        pltpu.SemaphoreType.DMA,
    ],
    compiler_params=pltpu.CompilerParams(needs_layout_passes=False),
)
def _emb(tab, idx, out, row_v, idx_v, dst_v, sem_r, sem_i, sem_o):
    wid = lax.axis_index("s") * NC + lax.axis_index("c")
    j0 = wid * RPW

    @pl.loop(0, RPW)
    def _row(r):
        j = j0 + r
        f = j // D
        d = lax.rem(j, D)
        cp_r = pltpu.async_copy(tab.at[f, d, pl.ds(0, V)], row_v, sem_r)
        cp_i = [None, None]
        cp_i[0] = pltpu.async_copy(idx.at[f, pl.ds(0, ICH)], idx_v.at[0], sem_i)

        @pl.when(r > 0)
        def _():
            pltpu.make_async_copy(dst_v, out.at[j, pl.ds(0, B)], sem_o).wait()

        cp_r.wait()
        for c in range(NCH):
            buf = c % 2
            cp_i[buf].wait()
            if c + 1 < NCH:
                cp_i[1 - buf] = pltpu.async_copy(
                    idx.at[f, pl.ds((c + 1) * ICH, ICH)], idx_v.at[1 - buf], sem_i
                )

            @plsc.parallel_loop(0, ICH // 16, unroll=32)
            def _g(t):
                iv = idx_v[buf, pl.ds(t * 16, 16)]
                dst_v[pl.ds(c * ICH + t * 16, 16)] = plsc.load_gather(row_v, [iv])

        pltpu.async_copy(dst_v, out.at[j, pl.ds(0, B)], sem_o)

    pltpu.make_async_copy(
        dst_v, out.at[NR - 1, pl.ds(0, B)], sem_o
    ).wait()


def kernel(index_list, tables):
    tab_t = tables.transpose(0, 2, 1)
    idx_t = index_list.astype(jnp.int32).T
    out_t = _emb(tab_t, idx_t)
    return out_t.T


# speedup vs baseline: 1.1706x; 1.1669x over previous
import functools

import jax
import jax.numpy as jnp
from jax import lax
from jax.experimental import pallas as pl
from jax.experimental.pallas import tpu as pltpu
from jax.experimental.pallas import tpu_sc as plsc

F = 26
V = 100000
D = 32
B = 16384
NC, NS = 2, 16
NW = NC * NS
NR = F * D
RPW = NR // NW
HB = B // 2

_mesh = plsc.VectorSubcoreMesh(
    core_axis_name="c", subcore_axis_name="s", num_cores=NC, num_subcores=NS
)


@functools.partial(
    pl.kernel,
    out_type=jax.ShapeDtypeStruct((NR, B), jnp.float32),
    mesh=_mesh,
    scratch_types=[
        pltpu.VMEM((V,), jnp.float32),
        pltpu.VMEM((B,), jnp.int32),
        pltpu.VMEM((HB,), jnp.float32),
        pltpu.SemaphoreType.DMA,
        pltpu.SemaphoreType.DMA,
        pltpu.SemaphoreType.DMA,
    ],
    compiler_params=pltpu.CompilerParams(needs_layout_passes=False),
)
def _emb(tab, idx, out, row_v, idx_v, dst_v, sem_r, sem_i, sem_o):
    wid = lax.axis_index("s") * NC + lax.axis_index("c")
    j0 = wid * RPW

    @pl.loop(0, RPW, init_carry=jnp.int32(-1))
    def _row(r, f_loaded):
        j = j0 + r
        f = j // D
        d = lax.rem(j, D)
        cp_r = pltpu.async_copy(tab.at[f, d, pl.ds(0, V)], row_v, sem_r)

        @pl.when(f != f_loaded)
        def _():
            pltpu.async_copy(idx.at[f], idx_v, sem_i).wait()

        cp_r.wait()
        for h in range(2):
            if h == 1:
                pltpu.make_async_copy(
                    dst_v, out.at[j, pl.ds(0, HB)], sem_o
                ).wait()
            else:
                @pl.when(r > 0)
                def _():
                    pltpu.make_async_copy(
                        dst_v, out.at[j, pl.ds(0, HB)], sem_o
                    ).wait()

            @plsc.parallel_loop(0, HB // 16, unroll=16)
            def _g(t):
                iv = idx_v[pl.ds(h * HB + t * 16, 16)]
                dst_v[pl.ds(t * 16, 16)] = plsc.load_gather(row_v, [iv])

            pltpu.async_copy(dst_v, out.at[j, pl.ds(h * HB, HB)], sem_o)

        return f

    pltpu.make_async_copy(dst_v, out.at[NR - 1, pl.ds(HB, HB)], sem_o).wait()


def kernel(index_list, tables):
    tab_t = tables.transpose(0, 2, 1)
    idx_t = index_list.astype(jnp.int32).T
    out_t = _emb(tab_t, idx_t)
    return out_t.T
